# baseline (speedup 1.0000x reference)
import functools

import jax
import jax.numpy as jnp
from jax import lax
from jax.experimental import pallas as pl
from jax.experimental.pallas import tpu as pltpu
from jax.experimental.pallas import tpu_sc as plsc

BATCH = 16384
HIST = 50
DIM = 16
NUM_ACT = 100000
TOTAL = BATCH * HIST
NUM_WORKERS = 32
PER_WORKER = TOTAL // NUM_WORKERS
CHUNK = 640
NCHUNKS = PER_WORKER // CHUNK
NBUF = 2
STAGE = NUM_ACT // 16

_mesh = plsc.VectorSubcoreMesh(core_axis_name="c", subcore_axis_name="s")


@functools.partial(
    pl.kernel,
    mesh=_mesh,
    out_type=jax.ShapeDtypeStruct((TOTAL, DIM), jnp.float32),
    scratch_types=[
        pltpu.VMEM_SHARED((NUM_ACT, DIM), jnp.float32),
        pltpu.VMEM((NBUF, CHUNK), jnp.int32),
        pltpu.VMEM((NBUF, CHUNK, DIM), jnp.float32),
        pltpu.SemaphoreType.DMA,
        pltpu.SemaphoreType.DMA,
        pltpu.SemaphoreType.DMA,
        pltpu.SemaphoreType.DMA,
    ],
    compiler_params=pltpu.CompilerParams(use_tc_tiling_on_sc=False),
)
def _gather_rows(idx_hbm, table_hbm, out_hbm, table_sp, idx_v, rows_v,
                 g0, g1, s0, s1):
    cid = lax.axis_index("c")
    sid = lax.axis_index("s")
    wid = sid * 2 + cid
    base = wid * PER_WORKER
    gsem = (g0, g1)
    ssem = (s0, s1)

    pltpu.sync_copy(table_hbm.at[pl.ds(sid * STAGE, STAGE)],
                    table_sp.at[pl.ds(sid * STAGE, STAGE)])
    plsc.subcore_barrier()

    def idx_load(g):
        b = g % NBUF
        pltpu.sync_copy(idx_hbm.at[pl.ds(base + g * CHUNK, CHUNK)],
                        idx_v.at[b])

    def gather_start(g):
        b = g % NBUF
        return pltpu.async_copy(
            table_sp.at[idx_v.at[b]], rows_v.at[b], gsem[b])

    def store_start(g):
        b = g % NBUF
        return pltpu.async_copy(
            rows_v.at[b], out_hbm.at[pl.ds(base + g * CHUNK, CHUNK)], ssem[b])

    idx_load(0)
    gh = {0: gather_start(0)}
    sh = {}
    for g in range(NCHUNKS):
        if g + 1 < NCHUNKS:
            if g >= 1:
                sh[g - 1].wait()
            idx_load(g + 1)
            gh[g + 1] = gather_start(g + 1)
        gh[g].wait()
        sh[g] = store_start(g)
    sh[NCHUNKS - 2].wait()
    sh[NCHUNKS - 1].wait()


def kernel(action_history, embedding_weight):
    idx = action_history.reshape(-1).astype(jnp.int32)
    out = _gather_rows(idx, embedding_weight)
    return out.reshape(action_history.shape[0], HIST * DIM)
